# baseline (speedup 1.0000x reference)
import functools

import jax
import jax.numpy as jnp
from jax import lax
from jax.experimental import pallas as pl
from jax.experimental.pallas import tpu as pltpu
from jax.experimental.pallas import tpu_sc as plsc

NUM_GRAPHS = 64
E_ROWS = 320000
N_ROWS = 10000
E_FEATS = 16
X_FEATS = 128
OUT_FEATS = 128

NC = 2
NS = 16
NW = NC * NS

E_PER_W = E_ROWS // NW
E_CH = 2000
E_NCH = E_PER_W // E_CH

N_CH = 80
N_NCH = N_ROWS // N_CH
N_ITERS = (N_NCH + NW - 1) // NW


def _sc_body(e_hbm, eids_hbm, x_hbm, nids_hbm, pe_hbm, px_hbm,
             ebuf, eids, eacc, nbuf, nids, nacc):
    wid = lax.axis_index("s") * NC + lax.axis_index("c")

    zero16 = jnp.zeros((16,), jnp.float32)

    def _zrow(g, _):
        eacc[g] = zero16
        for k in range(X_FEATS // 16):
            nacc[g, pl.ds(k * 16, 16)] = zero16
        return 0

    lax.fori_loop(0, NUM_GRAPHS, _zrow, 0)

    def _echunk(c, _):
        base = wid * E_PER_W + c * E_CH
        pltpu.sync_copy(e_hbm.at[pl.ds(base, E_CH)], ebuf)
        pltpu.sync_copy(eids_hbm.at[pl.ds(base, E_CH)], eids)

        def _egrp(g, _):
            r0 = g * 16
            idvec = eids[pl.ds(r0, 16)]
            for j in range(16):
                plsc.addupdate(eacc.at[idvec[j]], ebuf[r0 + j])
            return 0

        lax.fori_loop(0, E_CH // 16, _egrp, 0)
        return 0

    lax.fori_loop(0, E_NCH, _echunk, 0)

    def _nchunk(it, _):
        c = wid + it * NW

        @pl.when(c < N_NCH)
        def _do():
            base = c * N_CH
            pltpu.sync_copy(x_hbm.at[pl.ds(base, N_CH)], nbuf)
            pltpu.sync_copy(nids_hbm.at[pl.ds(base, N_CH)], nids)

            def _ngrp(g, _):
                r0 = g * 16
                idvec = nids[pl.ds(r0, 16)]
                for j in range(16):
                    gid = idvec[j]
                    for k in range(X_FEATS // 16):
                        plsc.addupdate(nacc.at[gid, pl.ds(k * 16, 16)],
                                       nbuf[r0 + j, pl.ds(k * 16, 16)])
                return 0

            lax.fori_loop(0, N_CH // 16, _ngrp, 0)

        return 0

    lax.fori_loop(0, N_ITERS, _nchunk, 0)

    pltpu.sync_copy(eacc, pe_hbm.at[wid])
    pltpu.sync_copy(nacc, px_hbm.at[wid])


_sc_segsum = pl.kernel(
    _sc_body,
    out_type=[
        jax.ShapeDtypeStruct((NW, NUM_GRAPHS, E_FEATS), jnp.float32),
        jax.ShapeDtypeStruct((NW, NUM_GRAPHS, X_FEATS), jnp.float32),
    ],
    mesh=plsc.VectorSubcoreMesh(core_axis_name="c", subcore_axis_name="s"),
    scratch_types=[
        pltpu.VMEM((E_CH, E_FEATS), jnp.float32),
        pltpu.VMEM((E_CH,), jnp.int32),
        pltpu.VMEM((NUM_GRAPHS, E_FEATS), jnp.float32),
        pltpu.VMEM((N_CH, X_FEATS), jnp.float32),
        pltpu.VMEM((N_CH,), jnp.int32),
        pltpu.VMEM((NUM_GRAPHS, X_FEATS), jnp.float32),
    ],
    compiler_params=pltpu.CompilerParams(use_tc_tiling_on_sc=False),
)


def _lin_body(pe_ref, px_ref, g_ref, w_ref, b_ref, out_ref):
    agg_e = jnp.sum(pe_ref[...], axis=0)
    agg_x = jnp.sum(px_ref[...], axis=0)
    w = w_ref[...]
    out = jax.lax.dot(agg_e, w[0:E_FEATS, :],
                      preferred_element_type=jnp.float32)
    out += jax.lax.dot(agg_x, w[E_FEATS:E_FEATS + X_FEATS, :],
                       preferred_element_type=jnp.float32)
    out += jax.lax.dot(g_ref[...], w[E_FEATS + X_FEATS:, :],
                       preferred_element_type=jnp.float32)
    out_ref[...] = out + b_ref[0, :][None, :]


def _linear(pe, px, global_attr, W, b2):
    return pl.pallas_call(
        _lin_body,
        out_shape=jax.ShapeDtypeStruct((NUM_GRAPHS, OUT_FEATS), jnp.float32),
    )(pe, px, global_attr, W, b2)


@jax.jit
def _global_block(x, e, global_attr, node_ids, edge_ids, W, b):
    pe, px = _sc_segsum(e, edge_ids.astype(jnp.int32),
                        x, node_ids.astype(jnp.int32))
    return _linear(pe, px, global_attr, W, b.reshape(1, OUT_FEATS))


def kernel(x, e, global_attr, node_graph_ids, edge_graph_ids, W, b):
    return _global_block(x, e, global_attr, node_graph_ids, edge_graph_ids,
                         W, b)


# speedup vs baseline: 1.2005x; 1.2005x over previous
import functools

import jax
import jax.numpy as jnp
from jax import lax
from jax.experimental import pallas as pl
from jax.experimental.pallas import tpu as pltpu
from jax.experimental.pallas import tpu_sc as plsc

NUM_GRAPHS = 64
E_ROWS = 320000
N_ROWS = 10000
E_FEATS = 16
X_FEATS = 128
OUT_FEATS = 128

NC = 2
NS = 16
NW = NC * NS

E_CH = 2560
E_GRP = E_CH // 128
E_NCH = E_ROWS // E_CH
N_CH = 80
N_NCH = N_ROWS // N_CH
ITERS = (E_NCH + NW - 1) // NW


def _sc_body(e_hbm, eids_hbm, x_hbm, nids_hbm, pe_hbm, px_hbm,
             ebuf, eids, eacc_v, nbuf, nids, nacc_v, eacc_sp, nacc_sp):
    cid = lax.axis_index("c")
    sid = lax.axis_index("s")
    wid = sid * NC + cid

    zero16 = jnp.zeros((16,), jnp.float32)

    def _zrow(g, _):
        eacc_v[g] = zero16
        for k in range(X_FEATS // 16):
            nacc_v[g, pl.ds(k * 16, 16)] = zero16
        return 0

    lax.fori_loop(0, NUM_GRAPHS, _zrow, 0)
    pltpu.sync_copy(eacc_v, eacc_sp.at[sid])
    pltpu.sync_copy(nacc_v, nacc_sp.at[sid])

    def _echunk(it, _):
        c = wid + it * NW

        @pl.when(c < E_NCH)
        def _do():
            pltpu.sync_copy(e_hbm.at[pl.ds(c * E_CH, E_CH)], ebuf)
            pltpu.sync_copy(eids_hbm.at[pl.ds(c * E_GRP, E_GRP)], eids)
            for j in range(E_GRP):
                pltpu.sync_copy(ebuf.at[pl.ds(j * 128, 128)],
                                eacc_sp.at[sid].at[eids.at[j]], add=True)

        return 0

    lax.fori_loop(0, ITERS, _echunk, 0)

    def _nchunk(it, _):
        c = wid + it * NW

        @pl.when(c < N_NCH)
        def _do():
            pltpu.sync_copy(x_hbm.at[pl.ds(c * N_CH, N_CH)], nbuf)
            pltpu.sync_copy(nids_hbm.at[pl.ds(c, 1)], nids)
            pltpu.sync_copy(nbuf, nacc_sp.at[sid].at[nids.at[0]], add=True)

        return 0

    lax.fori_loop(0, ITERS, _nchunk, 0)

    pltpu.sync_copy(eacc_sp.at[sid], pe_hbm.at[wid])
    pltpu.sync_copy(nacc_sp.at[sid], px_hbm.at[wid])


_sc_segsum = pl.kernel(
    _sc_body,
    out_type=[
        jax.ShapeDtypeStruct((NW, NUM_GRAPHS, E_FEATS), jnp.float32),
        jax.ShapeDtypeStruct((NW, NUM_GRAPHS, X_FEATS), jnp.float32),
    ],
    mesh=plsc.VectorSubcoreMesh(core_axis_name="c", subcore_axis_name="s"),
    scratch_types=[
        pltpu.VMEM((E_CH, E_FEATS), jnp.float32),
        pltpu.VMEM((E_GRP, 128), jnp.int32),
        pltpu.VMEM((NUM_GRAPHS, E_FEATS), jnp.float32),
        pltpu.VMEM((N_CH, X_FEATS), jnp.float32),
        pltpu.VMEM((1, N_CH), jnp.int32),
        pltpu.VMEM((NUM_GRAPHS, X_FEATS), jnp.float32),
        pltpu.VMEM_SHARED((NS, NUM_GRAPHS, E_FEATS), jnp.float32),
        pltpu.VMEM_SHARED((NS, NUM_GRAPHS, X_FEATS), jnp.float32),
    ],
    compiler_params=pltpu.CompilerParams(use_tc_tiling_on_sc=False),
)


def _lin_body(pe_ref, px_ref, g_ref, w_ref, b_ref, out_ref):
    agg_e = jnp.sum(pe_ref[...], axis=0)
    agg_x = jnp.sum(px_ref[...], axis=0)
    w = w_ref[...]
    out = jax.lax.dot(agg_e, w[0:E_FEATS, :],
                      preferred_element_type=jnp.float32)
    out += jax.lax.dot(agg_x, w[E_FEATS:E_FEATS + X_FEATS, :],
                       preferred_element_type=jnp.float32)
    out += jax.lax.dot(g_ref[...], w[E_FEATS + X_FEATS:, :],
                       preferred_element_type=jnp.float32)
    out_ref[...] = out + b_ref[0, :][None, :]


def _linear(pe, px, global_attr, W, b2):
    return pl.pallas_call(
        _lin_body,
        out_shape=jax.ShapeDtypeStruct((NUM_GRAPHS, OUT_FEATS), jnp.float32),
    )(pe, px, global_attr, W, b2)


@jax.jit
def _global_block(x, e, global_attr, node_ids, edge_ids, W, b):
    eids2 = edge_ids.astype(jnp.int32).reshape(E_ROWS // 128, 128)
    nids2 = node_ids.astype(jnp.int32).reshape(N_NCH, N_CH)
    pe, px = _sc_segsum(e, eids2, x, nids2)
    return _linear(pe, px, global_attr, W, b.reshape(1, OUT_FEATS))


def kernel(x, e, global_attr, node_graph_ids, edge_graph_ids, W, b):
    return _global_block(x, e, global_attr, node_graph_ids, edge_graph_ids,
                         W, b)


# speedup vs baseline: 2.8639x; 2.3855x over previous
import functools

import jax
import jax.numpy as jnp
from jax import lax
from jax.experimental import pallas as pl
from jax.experimental.pallas import tpu as pltpu
from jax.experimental.pallas import tpu_sc as plsc

NUM_GRAPHS = 64
E_ROWS = 320000
N_ROWS = 10000
E_FEATS = 16
X_FEATS = 128
OUT_FEATS = 128

NC = 2
NS = 16
NW = NC * NS

N_CH = 80
N_NCH = N_ROWS // N_CH
N_ITERS = (N_NCH + NW - 1) // NW

E_GRID = 50
E_BLK = E_ROWS // E_GRID


def _sc_body(x_hbm, nids_hbm, px_hbm, nbuf, nids, zbuf, nacc_sp):
    cid = lax.axis_index("c")
    sid = lax.axis_index("s")
    wid = sid * NC + cid

    zero16 = jnp.zeros((16,), jnp.float32)

    def _zrow(g, _):
        for k in range(X_FEATS // 16):
            zbuf[g, pl.ds(k * 16, 16)] = zero16
        return 0

    lax.fori_loop(0, NUM_GRAPHS, _zrow, 0)
    pltpu.sync_copy(zbuf, nacc_sp.at[sid])

    def _nchunk(it, _):
        c = wid + it * NW

        @pl.when(c < N_NCH)
        def _do():
            pltpu.sync_copy(x_hbm.at[pl.ds(c * N_CH, N_CH)], nbuf)
            pltpu.sync_copy(nids_hbm.at[pl.ds(c, 1)], nids)
            pltpu.sync_copy(nbuf, nacc_sp.at[sid].at[nids.at[0]], add=True)

        return 0

    lax.fori_loop(0, N_ITERS, _nchunk, 0)

    pltpu.sync_copy(nacc_sp.at[sid], px_hbm.at[wid])


_sc_nodes = pl.kernel(
    _sc_body,
    out_type=jax.ShapeDtypeStruct((NW, NUM_GRAPHS, X_FEATS), jnp.float32),
    mesh=plsc.VectorSubcoreMesh(core_axis_name="c", subcore_axis_name="s"),
    scratch_types=[
        pltpu.VMEM((N_CH, X_FEATS), jnp.float32),
        pltpu.VMEM((1, N_CH), jnp.int32),
        pltpu.VMEM((NUM_GRAPHS, X_FEATS), jnp.float32),
        pltpu.VMEM_SHARED((NS, NUM_GRAPHS, X_FEATS), jnp.float32),
    ],
    compiler_params=pltpu.CompilerParams(use_tc_tiling_on_sc=False),
)


def _edges_body(eids_ref, et_ref, acc_ref, acc):
    step = pl.program_id(0)

    @pl.when(step == 0)
    def _init():
        acc[...] = jnp.zeros_like(acc)

    gids = lax.broadcasted_iota(jnp.int32, (E_BLK, NUM_GRAPHS), 1)
    onehot = (eids_ref[0, 0, :][:, None] == gids).astype(jnp.float32)
    acc[...] += jax.lax.dot(et_ref[...], onehot,
                            preferred_element_type=jnp.float32)

    @pl.when(step == E_GRID - 1)
    def _done():
        acc_ref[...] = acc[...]


def _tc_edges(et, eids3):
    return pl.pallas_call(
        _edges_body,
        grid=(E_GRID,),
        in_specs=[
            pl.BlockSpec((1, 1, E_BLK), lambda i: (i, 0, 0)),
            pl.BlockSpec((E_FEATS, E_BLK), lambda i: (0, i)),
        ],
        out_specs=pl.BlockSpec((E_FEATS, NUM_GRAPHS), lambda i: (0, 0)),
        out_shape=jax.ShapeDtypeStruct((E_FEATS, NUM_GRAPHS), jnp.float32),
        scratch_shapes=[pltpu.VMEM((E_FEATS, NUM_GRAPHS), jnp.float32)],
        compiler_params=pltpu.CompilerParams(
            dimension_semantics=("arbitrary",),
        ),
    )(eids3, et)


def _lin_body(aet_ref, px_ref, g_ref, w_ref, b_ref, out_ref):
    agg_x = jnp.sum(px_ref[...], axis=0)
    w = w_ref[...]
    out = jax.lax.dot_general(aet_ref[...], w[0:E_FEATS, :],
                              (((0,), (0,)), ((), ())),
                              preferred_element_type=jnp.float32)
    out += jax.lax.dot(agg_x, w[E_FEATS:E_FEATS + X_FEATS, :],
                       preferred_element_type=jnp.float32)
    out += jax.lax.dot(g_ref[...], w[E_FEATS + X_FEATS:, :],
                       preferred_element_type=jnp.float32)
    out_ref[...] = out + b_ref[0, :][None, :]


def _linear(agg_et, px, global_attr, W, b2):
    return pl.pallas_call(
        _lin_body,
        out_shape=jax.ShapeDtypeStruct((NUM_GRAPHS, OUT_FEATS), jnp.float32),
    )(agg_et, px, global_attr, W, b2)


@jax.jit
def _global_block(x, e, global_attr, node_ids, edge_ids, W, b):
    et = e.T
    eids3 = edge_ids.astype(jnp.int32).reshape(E_GRID, 1, E_BLK)
    nids2 = node_ids.astype(jnp.int32).reshape(N_NCH, N_CH)
    px = _sc_nodes(x, nids2)
    agg_et = _tc_edges(et, eids3)
    return _linear(agg_et, px, global_attr, W, b.reshape(1, OUT_FEATS))


def kernel(x, e, global_attr, node_graph_ids, edge_graph_ids, W, b):
    return _global_block(x, e, global_attr, node_graph_ids, edge_graph_ids,
                         W, b)
